# baseline (speedup 1.0000x reference)
import functools

import jax
import jax.numpy as jnp
from jax import lax
from jax.experimental import pallas as pl
from jax.experimental.pallas import tpu as pltpu
from jax.experimental.pallas import tpu_sc as plsc

_N = 10000
_D = 128
_E = 320000
_EPS = 1e-5

_NC = 2
_NS = 16
_NT = _NC * _NS
_CH = 64
_CPT = 160
_EPT = _CPT * _CH
_EPAD = _NT * _EPT
_SLOTS = 4
_NROW = 10112
_RPT = _NROW // _NS
_NPAD = 10112
_NHI = 128


def _edge_weights(edge_attr, w0, b0, w1, b1):
    bE = 10000

    def kern(ea_ref, w0_ref, b0_ref, w1_ref, b1_ref, out_ref):
        ea = ea_ref[...]
        z0 = jnp.dot(ea, w0_ref[...], preferred_element_type=jnp.float32) + b0_ref[...]
        z1 = jnp.dot(ea, w1_ref[...], preferred_element_type=jnp.float32) + b1_ref[...]
        z = jnp.concatenate([z0, z1], axis=1)
        out_ref[...] = jnp.maximum(z, 0.0) + jnp.log1p(jnp.exp(-jnp.abs(z)))

    return pl.pallas_call(
        kern,
        grid=(_E // bE,),
        in_specs=[
            pl.BlockSpec((bE, 16), lambda i: (i, 0)),
            pl.BlockSpec((16, 1), lambda i: (0, 0)),
            pl.BlockSpec((1, 1), lambda i: (0, 0)),
            pl.BlockSpec((16, 1), lambda i: (0, 0)),
            pl.BlockSpec((1, 1), lambda i: (0, 0)),
        ],
        out_specs=pl.BlockSpec((bE, 2), lambda i: (i, 0)),
        out_shape=jax.ShapeDtypeStruct((_E, 2), jnp.float32),
    )(edge_attr, w0, b0.reshape(1, 1), w1, b1.reshape(1, 1))


def _sc_scatter(x, pck_t, wpk_t):
    mesh = plsc.VectorSubcoreMesh(core_axis_name="c", subcore_axis_name="s")

    @functools.partial(
        pl.kernel,
        mesh=mesh,
        out_type=jax.ShapeDtypeStruct((_NC, _NROW, _D), jnp.float32),
        scratch_types=[
            pltpu.VMEM((_CPT // 2, 128), jnp.int32),
            pltpu.VMEM((_CPT // 4, 128), jnp.int32),
            pltpu.VMEM((2 * _CH,), jnp.int32),
            pltpu.VMEM((2, _CH), jnp.int32),
            pltpu.VMEM((2 * _CH,), jnp.float32),
            pltpu.VMEM((2, _CH, _D), jnp.float32),
            pltpu.VMEM_SHARED((_NROW, _D), jnp.float32),
            pltpu.SemaphoreType.DMA,
            pltpu.SemaphoreType.DMA,
        ],
    )
    def k(x_hbm, pck_hbm, wpk_hbm, out_hbm,
          pckb, wpkb, sidx, didx, wf, rowsb, acc_s, gsem, ssem):
        cid = lax.axis_index("c")
        sid = lax.axis_index("s")
        wid = cid * _NS + sid

        zero16 = jnp.zeros((16,), jnp.float32)

        def zrow(r, c):
            for j in range(_D // 16):
                rowsb[0, r, pl.ds(j * 16, 16)] = zero16
            return c
        lax.fori_loop(0, _CH, zrow, 0)

        nz = _RPT // _CH
        for t in range(nz):
            pltpu.async_copy(rowsb.at[0],
                             acc_s.at[pl.ds(sid * _RPT + t * _CH, _CH)],
                             ssem)
        rem = _RPT % _CH
        if rem:
            pltpu.async_copy(
                rowsb.at[0, pl.ds(0, rem)],
                acc_s.at[pl.ds(sid * _RPT + nz * _CH, rem)], ssem)
        for t in range(nz):
            pltpu.make_async_copy(
                rowsb.at[0], acc_s.at[pl.ds(sid * _RPT, _CH)], ssem).wait()
        if rem:
            pltpu.make_async_copy(
                rowsb.at[0, pl.ds(0, rem)],
                acc_s.at[pl.ds(sid * _RPT, rem)], ssem).wait()
        plsc.subcore_barrier()

        pltpu.sync_copy(pck_hbm.at[wid], pckb)
        pltpu.sync_copy(wpk_hbm.at[wid], wpkb)

        def unpack_idx(row, colbase, slot):
            for u in range(_CH // 16):
                pv = pckb[row, pl.ds(colbase + u * 16, 16)]
                sidx[pl.ds(slot * _CH + u * 16, 16)] = pv & 16383
                didx[slot, pl.ds(u * 16, 16)] = pv >> 14

        def gidx(slot):
            return sidx.at[pl.ds(slot * _CH, _CH)]

        unpack_idx(0, 0, 0)
        pltpu.async_copy(x_hbm.at[gidx(0)], rowsb.at[0], gsem)

        def quad(p, c):
            for b in range(4):
                i = p * 4 + b
                s = b % 2
                so = (b + 1) % 2

                pltpu.make_async_copy(
                    x_hbm.at[gidx(s)], rowsb.at[s], gsem).wait()

                for q in range(_CH // 32):
                    wv = wpkb[p, pl.ds(b * (_CH // 2) + q * 16, 16)]
                    wf[pl.ds(s * _CH + q * 32, 16)] = (
                        lax.bitcast_convert_type(wv << 16, jnp.float32))
                    wf[pl.ds(s * _CH + q * 32 + 16, 16)] = (
                        lax.bitcast_convert_type(wv & jnp.int32(-65536),
                                                 jnp.float32))

                def grp(g, c2):
                    wv = wf[pl.ds(s * _CH + g * 16, 16)]
                    for kk in range(16):
                        ws = wv[kk]
                        e = g * 16 + kk
                        for j in range(_D // 16):
                            sl = pl.ds(j * 16, 16)
                            rowsb[s, e, sl] = rowsb[s, e, sl] * ws
                    return c2
                lax.fori_loop(0, _CH // 16, grp, 0)

                pltpu.async_copy(rowsb.at[s], acc_s.at[didx.at[s]],
                                 ssem, add=True)

                @pl.when(i >= 1)
                def _():
                    pltpu.make_async_copy(
                        rowsb.at[so], acc_s.at[didx.at[so]], ssem).wait()

                @pl.when(i + 1 < _CPT)
                def _():
                    unpack_idx(2 * p + (b + 1) // 2, ((b + 1) % 2) * _CH, so)
                    pltpu.async_copy(
                        x_hbm.at[gidx(so)], rowsb.at[so], gsem)
            return c
        lax.fori_loop(0, _CPT // 4, quad, 0)

        pltpu.make_async_copy(
            rowsb.at[(_CPT - 1) % 2], acc_s.at[didx.at[(_CPT - 1) % 2]],
            ssem).wait()

        plsc.subcore_barrier()
        pltpu.sync_copy(acc_s.at[pl.ds(sid * _RPT, _RPT)],
                        out_hbm.at[cid, pl.ds(sid * _RPT, _RPT)])

    return k(x, pck_t, wpk_t)


def _deg_weights(dst3, w01):
    bE = 16000
    nb = _E // bE

    def kern(dst_ref, w_ref, out_ref):
        d = dst_ref[...].reshape(bE, 1)
        hi = d >> 7
        lo = d & 127
        cols = lax.broadcasted_iota(jnp.int32, (bE, _NHI), 1)
        a = jnp.where(hi == cols, 1.0, 0.0)
        ohlo = jnp.where(lo == cols, 1.0, 0.0)
        w = w_ref[...]
        b0 = ohlo * w[:, 0:1]
        b1 = ohlo * w[:, 1:2]
        g0 = lax.dot_general(a, b0, (((0,), (0,)), ((), ())),
                             preferred_element_type=jnp.float32)
        g1 = lax.dot_general(a, b1, (((0,), (0,)), ((), ())),
                             preferred_element_type=jnp.float32)
        g = jnp.stack([g0, g1])

        @pl.when(pl.program_id(0) == 0)
        def _():
            out_ref[...] = g

        @pl.when(pl.program_id(0) != 0)
        def _():
            out_ref[...] += g

    return pl.pallas_call(
        kern,
        grid=(nb,),
        in_specs=[
            pl.BlockSpec((1, 1, bE), lambda i: (i, 0, 0)),
            pl.BlockSpec((bE, 2), lambda i: (i, 0)),
        ],
        out_specs=pl.BlockSpec((2, _NHI, 128), lambda i: (0, 0, 0)),
        out_shape=jax.ShapeDtypeStruct((2, _NHI, 128), jnp.float32),
    )(dst3, w01)


def _combine(p0, p1, dw, xin, lin_W, lin_b, ln_g, ln_bt,
             fc_W=None, fc_b=None):
    bN = 1000
    final = fc_W is not None

    def kern(*refs):
        if final:
            (p0_ref, p1_ref, dw_ref, x_ref, w_ref, b_ref,
             g_ref, bt_ref, fw_ref, fb_ref, out_ref) = refs
        else:
            (p0_ref, p1_ref, dw_ref, x_ref, w_ref, b_ref,
             g_ref, bt_ref, out_ref) = refs
        aggr = p0_ref[...] + p1_ref[...] - dw_ref[...] * x_ref[...]
        h = lax.dot_general(aggr, w_ref[...], (((1,), (1,)), ((), ())),
                            preferred_element_type=jnp.float32) + b_ref[...]
        h = jnp.maximum(h, 0.0)
        mu = jnp.mean(h, axis=1, keepdims=True)
        hc = h - mu
        var = jnp.mean(hc * hc, axis=1, keepdims=True)
        hn = hc * lax.rsqrt(var + _EPS) * g_ref[...] + bt_ref[...]
        if final:
            hn = lax.dot_general(hn, fw_ref[...], (((1,), (1,)), ((), ())),
                                 preferred_element_type=jnp.float32) + fb_ref[...]
        out_ref[...] = hn

    row = pl.BlockSpec((bN, _D), lambda i: (i, 0))
    col = pl.BlockSpec((bN, 1), lambda i: (i, 0))
    full = pl.BlockSpec((_D, _D), lambda i: (0, 0))
    vec = pl.BlockSpec((1, _D), lambda i: (0, 0))
    in_specs = [row, row, col, row, full, vec, vec, vec]
    args = [p0, p1, dw, xin, lin_W, lin_b.reshape(1, _D),
            ln_g.reshape(1, _D), ln_bt.reshape(1, _D)]
    if final:
        in_specs += [full, vec]
        args += [fc_W, fc_b.reshape(1, _D)]

    return pl.pallas_call(
        kern,
        grid=(_N // bN,),
        in_specs=in_specs,
        out_specs=row,
        out_shape=jax.ShapeDtypeStruct((_N, _D), jnp.float32),
    )(*args)


def kernel(x, edge_index, edge_attr, lin0_W, lin0_b, emlp0_W, emlp0_b,
           ln0_g, ln0_bt, lin1_W, lin1_b, emlp1_W, emlp1_b, ln1_g, ln1_bt,
           fc_W, fc_b):
    src = edge_index[0]
    dst = edge_index[1]

    w01 = _edge_weights(edge_attr, emlp0_W, emlp0_b, emlp1_W, emlp1_b)

    pad = _EPAD - _E
    pck = dst * 16384 + src
    pck_t = jnp.pad(pck, (0, pad)).reshape(_NT, _CPT // 2, 128)

    def pack_w(w):
        wt = jnp.pad(w, (0, pad)).reshape(_NT, _EPT // 32, 2, 16)
        bits = lax.bitcast_convert_type(
            wt.astype(jnp.bfloat16), jnp.uint16).astype(jnp.uint32)
        words = bits[:, :, 0, :] | (bits[:, :, 1, :] << 16)
        return lax.bitcast_convert_type(words, jnp.int32).reshape(
            _NT, _CPT // 4, 128)

    w0_t = pack_w(w01[:, 0])
    w1_t = pack_w(w01[:, 1])

    dwg = _deg_weights(dst.reshape(_E // 16000, 1, 16000), w01)
    dw0 = dwg[0].reshape(_NHI * 128)[:_N].reshape(_N, 1)
    dw1 = dwg[1].reshape(_NHI * 128)[:_N].reshape(_N, 1)

    p = _sc_scatter(x, pck_t, w0_t)
    h = _combine(p[0, :_N], p[1, :_N], dw0, x, lin0_W, lin0_b, ln0_g, ln0_bt)

    p = _sc_scatter(h, pck_t, w1_t)
    return _combine(p[0, :_N], p[1, :_N], dw1, h, lin1_W, lin1_b,
                    ln1_g, ln1_bt, fc_W, fc_b)


# speedup vs baseline: 1.0895x; 1.0895x over previous
import functools

import jax
import jax.numpy as jnp
from jax import lax
from jax.experimental import pallas as pl
from jax.experimental.pallas import tpu as pltpu
from jax.experimental.pallas import tpu_sc as plsc

_N = 10000
_D = 128
_E = 320000
_EPS = 1e-5

_NC = 2
_NS = 16
_NT = _NC * _NS
_CH = 128
_CPT = 80
_EPT = _CPT * _CH
_EPAD = _NT * _EPT
_SLOTS = 4
_NROW = 10240
_RPT = _NROW // _NS
_NPAD = 10112
_NHI = 128


def _edge_weights(edge_attr, w0, b0, w1, b1):
    bE = 10000

    def kern(ea_ref, w0_ref, b0_ref, w1_ref, b1_ref, out_ref):
        ea = ea_ref[...]
        z0 = jnp.dot(ea, w0_ref[...], preferred_element_type=jnp.float32) + b0_ref[...]
        z1 = jnp.dot(ea, w1_ref[...], preferred_element_type=jnp.float32) + b1_ref[...]
        z = jnp.concatenate([z0, z1], axis=1)
        out_ref[...] = jnp.maximum(z, 0.0) + jnp.log1p(jnp.exp(-jnp.abs(z)))

    return pl.pallas_call(
        kern,
        grid=(_E // bE,),
        in_specs=[
            pl.BlockSpec((bE, 16), lambda i: (i, 0)),
            pl.BlockSpec((16, 1), lambda i: (0, 0)),
            pl.BlockSpec((1, 1), lambda i: (0, 0)),
            pl.BlockSpec((16, 1), lambda i: (0, 0)),
            pl.BlockSpec((1, 1), lambda i: (0, 0)),
        ],
        out_specs=pl.BlockSpec((bE, 2), lambda i: (i, 0)),
        out_shape=jax.ShapeDtypeStruct((_E, 2), jnp.float32),
    )(edge_attr, w0, b0.reshape(1, 1), w1, b1.reshape(1, 1))


def _sc_scatter(x, pck_t, wpk_t):
    mesh = plsc.VectorSubcoreMesh(core_axis_name="c", subcore_axis_name="s")

    @functools.partial(
        pl.kernel,
        mesh=mesh,
        out_type=jax.ShapeDtypeStruct((_NC, _NROW, _D), jnp.float32),
        scratch_types=[
            pltpu.VMEM((_CPT, 128), jnp.int32),
            pltpu.VMEM((_CPT // 2, 128), jnp.int32),
            pltpu.VMEM((2 * _CH,), jnp.int32),
            pltpu.VMEM((2, _CH), jnp.int32),
            pltpu.VMEM((2 * _CH,), jnp.float32),
            pltpu.VMEM((2, _CH, _D), jnp.float32),
            pltpu.VMEM_SHARED((_NROW, _D), jnp.float32),
            pltpu.SemaphoreType.DMA,
            pltpu.SemaphoreType.DMA,
        ],
    )
    def k(x_hbm, pck_hbm, wpk_hbm, out_hbm,
          pckb, wpkb, sidx, didx, wf, rowsb, acc_s, gsem, ssem):
        cid = lax.axis_index("c")
        sid = lax.axis_index("s")
        wid = cid * _NS + sid

        zero16 = jnp.zeros((16,), jnp.float32)

        def zrow(r, c):
            for j in range(_D // 16):
                rowsb[0, r, pl.ds(j * 16, 16)] = zero16
            return c
        lax.fori_loop(0, _CH, zrow, 0)

        nz = _RPT // _CH
        for t in range(nz):
            pltpu.async_copy(rowsb.at[0],
                             acc_s.at[pl.ds(sid * _RPT + t * _CH, _CH)],
                             ssem)
        for t in range(nz):
            pltpu.make_async_copy(
                rowsb.at[0], acc_s.at[pl.ds(sid * _RPT, _CH)], ssem).wait()
        plsc.subcore_barrier()

        pltpu.sync_copy(pck_hbm.at[wid], pckb)
        pltpu.sync_copy(wpk_hbm.at[wid], wpkb)

        def unpack_idx(row, slot):
            for u in range(_CH // 16):
                pv = pckb[row, pl.ds(u * 16, 16)]
                sidx[pl.ds(slot * _CH + u * 16, 16)] = pv & 16383
                didx[slot, pl.ds(u * 16, 16)] = pv >> 14

        def gidx(slot):
            return sidx.at[pl.ds(slot * _CH, _CH)]

        unpack_idx(0, 0)
        pltpu.async_copy(x_hbm.at[gidx(0)], rowsb.at[0], gsem)

        def quad(p, c):
            for b in range(4):
                i = p * 4 + b
                s = b % 2
                so = (b + 1) % 2

                pltpu.make_async_copy(
                    x_hbm.at[gidx(s)], rowsb.at[s], gsem).wait()

                @pl.when(i >= 1)
                def _():
                    pltpu.make_async_copy(
                        rowsb.at[so], acc_s.at[didx.at[so]], ssem).wait()

                @pl.when(i + 1 < _CPT)
                def _():
                    unpack_idx(i + 1, so)
                    pltpu.async_copy(
                        x_hbm.at[gidx(so)], rowsb.at[so], gsem)

                for q in range(_CH // 32):
                    wv = wpkb[2 * p + b // 2,
                              pl.ds((b % 2) * 64 + q * 16, 16)]
                    wf[pl.ds(s * _CH + q * 32, 16)] = (
                        lax.bitcast_convert_type(wv << 16, jnp.float32))
                    wf[pl.ds(s * _CH + q * 32 + 16, 16)] = (
                        lax.bitcast_convert_type(wv & jnp.int32(-65536),
                                                 jnp.float32))

                def grp(g, c2):
                    wv = wf[pl.ds(s * _CH + g * 16, 16)]
                    for kk in range(16):
                        ws = wv[kk]
                        e = g * 16 + kk
                        for j in range(_D // 16):
                            sl = pl.ds(j * 16, 16)
                            rowsb[s, e, sl] = rowsb[s, e, sl] * ws
                    return c2
                lax.fori_loop(0, _CH // 16, grp, 0)

                pltpu.async_copy(rowsb.at[s], acc_s.at[didx.at[s]],
                                 ssem, add=True)
            return c
        lax.fori_loop(0, _CPT // 4, quad, 0)

        pltpu.make_async_copy(
            rowsb.at[(_CPT - 1) % 2], acc_s.at[didx.at[(_CPT - 1) % 2]],
            ssem).wait()

        plsc.subcore_barrier()
        pltpu.sync_copy(acc_s.at[pl.ds(sid * _RPT, _RPT)],
                        out_hbm.at[cid, pl.ds(sid * _RPT, _RPT)])

    return k(x, pck_t, wpk_t)


def _deg_weights(dst3, w01):
    bE = 16000
    nb = _E // bE

    def kern(dst_ref, w_ref, out_ref):
        d = dst_ref[...].reshape(bE, 1)
        hi = d >> 7
        lo = d & 127
        cols = lax.broadcasted_iota(jnp.int32, (bE, _NHI), 1)
        a = jnp.where(hi == cols, 1.0, 0.0)
        ohlo = jnp.where(lo == cols, 1.0, 0.0)
        w = w_ref[...]
        b0 = ohlo * w[:, 0:1]
        b1 = ohlo * w[:, 1:2]
        g0 = lax.dot_general(a, b0, (((0,), (0,)), ((), ())),
                             preferred_element_type=jnp.float32)
        g1 = lax.dot_general(a, b1, (((0,), (0,)), ((), ())),
                             preferred_element_type=jnp.float32)
        g = jnp.stack([g0, g1])

        @pl.when(pl.program_id(0) == 0)
        def _():
            out_ref[...] = g

        @pl.when(pl.program_id(0) != 0)
        def _():
            out_ref[...] += g

    return pl.pallas_call(
        kern,
        grid=(nb,),
        in_specs=[
            pl.BlockSpec((1, 1, bE), lambda i: (i, 0, 0)),
            pl.BlockSpec((bE, 2), lambda i: (i, 0)),
        ],
        out_specs=pl.BlockSpec((2, _NHI, 128), lambda i: (0, 0, 0)),
        out_shape=jax.ShapeDtypeStruct((2, _NHI, 128), jnp.float32),
    )(dst3, w01)


def _combine(p0, p1, dw, xin, lin_W, lin_b, ln_g, ln_bt,
             fc_W=None, fc_b=None):
    bN = 1000
    final = fc_W is not None

    def kern(*refs):
        if final:
            (p0_ref, p1_ref, dw_ref, x_ref, w_ref, b_ref,
             g_ref, bt_ref, fw_ref, fb_ref, out_ref) = refs
        else:
            (p0_ref, p1_ref, dw_ref, x_ref, w_ref, b_ref,
             g_ref, bt_ref, out_ref) = refs
        aggr = p0_ref[...] + p1_ref[...] - dw_ref[...] * x_ref[...]
        h = lax.dot_general(aggr, w_ref[...], (((1,), (1,)), ((), ())),
                            preferred_element_type=jnp.float32) + b_ref[...]
        h = jnp.maximum(h, 0.0)
        mu = jnp.mean(h, axis=1, keepdims=True)
        hc = h - mu
        var = jnp.mean(hc * hc, axis=1, keepdims=True)
        hn = hc * lax.rsqrt(var + _EPS) * g_ref[...] + bt_ref[...]
        if final:
            hn = lax.dot_general(hn, fw_ref[...], (((1,), (1,)), ((), ())),
                                 preferred_element_type=jnp.float32) + fb_ref[...]
        out_ref[...] = hn

    row = pl.BlockSpec((bN, _D), lambda i: (i, 0))
    col = pl.BlockSpec((bN, 1), lambda i: (i, 0))
    full = pl.BlockSpec((_D, _D), lambda i: (0, 0))
    vec = pl.BlockSpec((1, _D), lambda i: (0, 0))
    in_specs = [row, row, col, row, full, vec, vec, vec]
    args = [p0, p1, dw, xin, lin_W, lin_b.reshape(1, _D),
            ln_g.reshape(1, _D), ln_bt.reshape(1, _D)]
    if final:
        in_specs += [full, vec]
        args += [fc_W, fc_b.reshape(1, _D)]

    return pl.pallas_call(
        kern,
        grid=(_N // bN,),
        in_specs=in_specs,
        out_specs=row,
        out_shape=jax.ShapeDtypeStruct((_N, _D), jnp.float32),
    )(*args)


def kernel(x, edge_index, edge_attr, lin0_W, lin0_b, emlp0_W, emlp0_b,
           ln0_g, ln0_bt, lin1_W, lin1_b, emlp1_W, emlp1_b, ln1_g, ln1_bt,
           fc_W, fc_b):
    src = edge_index[0]
    dst = edge_index[1]

    w01 = _edge_weights(edge_attr, emlp0_W, emlp0_b, emlp1_W, emlp1_b)

    pad = _EPAD - _E
    pck = dst * 16384 + src
    pck_t = jnp.pad(pck, (0, pad)).reshape(_NT, _CPT, 128)

    def pack_w(w):
        wt = jnp.pad(w, (0, pad)).reshape(_NT, _EPT // 32, 2, 16)
        bits = lax.bitcast_convert_type(
            wt.astype(jnp.bfloat16), jnp.uint16).astype(jnp.uint32)
        words = bits[:, :, 0, :] | (bits[:, :, 1, :] << 16)
        return lax.bitcast_convert_type(words, jnp.int32).reshape(
            _NT, _CPT // 2, 128)

    w0_t = pack_w(w01[:, 0])
    w1_t = pack_w(w01[:, 1])

    dwg = _deg_weights(dst.reshape(_E // 16000, 1, 16000), w01)
    dw0 = dwg[0].reshape(_NHI * 128)[:_N].reshape(_N, 1)
    dw1 = dwg[1].reshape(_NHI * 128)[:_N].reshape(_N, 1)

    p = _sc_scatter(x, pck_t, w0_t)
    h = _combine(p[0, :_N], p[1, :_N], dw0, x, lin0_W, lin0_b, ln0_g, ln0_bt)

    p = _sc_scatter(h, pck_t, w1_t)
    return _combine(p[0, :_N], p[1, :_N], dw1, h, lin1_W, lin1_b,
                    ln1_g, ln1_bt, fc_W, fc_b)


# speedup vs baseline: 1.4463x; 1.3275x over previous
import functools

import jax
import jax.numpy as jnp
from jax import lax
from jax.experimental import pallas as pl
from jax.experimental.pallas import tpu as pltpu
from jax.experimental.pallas import tpu_sc as plsc

_N = 10000
_D = 128
_E = 320000
_EPS = 1e-5

_NC = 2
_NS = 16
_NT = _NC * _NS
_CH = 128
_CPT = -(-_E // (_NT * _CH))
_EPT = _CPT * _CH
_EPAD = _NT * _EPT
_NROW = 10240
_RPT = _NROW // _NS
_ZR = 128
_NPAD = 10240
_DWPT = _NPAD // _NS


def _edge_weights(edge_attr, w0, b0, w1, b1):
    bE = 10000

    def kern(ea_ref, w0_ref, b0_ref, w1_ref, b1_ref, out_ref):
        ea = ea_ref[...]
        z0 = jnp.dot(ea, w0_ref[...], preferred_element_type=jnp.float32) + b0_ref[...]
        z1 = jnp.dot(ea, w1_ref[...], preferred_element_type=jnp.float32) + b1_ref[...]
        z = jnp.concatenate([z0, z1], axis=1)
        out_ref[...] = jnp.maximum(z, 0.0) + jnp.log1p(jnp.exp(-jnp.abs(z)))

    return pl.pallas_call(
        kern,
        grid=(_E // bE,),
        in_specs=[
            pl.BlockSpec((bE, 16), lambda i: (i, 0)),
            pl.BlockSpec((16, 1), lambda i: (0, 0)),
            pl.BlockSpec((1, 1), lambda i: (0, 0)),
            pl.BlockSpec((16, 1), lambda i: (0, 0)),
            pl.BlockSpec((1, 1), lambda i: (0, 0)),
        ],
        out_specs=pl.BlockSpec((bE, 2), lambda i: (i, 0)),
        out_shape=jax.ShapeDtypeStruct((_E, 2), jnp.float32),
    )(edge_attr, w0, b0.reshape(1, 1), w1, b1.reshape(1, 1))


def _sc_scatter(x, src_t, dst_t, w_t):
    mesh = plsc.VectorSubcoreMesh(core_axis_name="c", subcore_axis_name="s")

    @functools.partial(
        pl.kernel,
        mesh=mesh,
        out_type=(
            jax.ShapeDtypeStruct((_NC, _NROW, _D), jnp.float32),
            jax.ShapeDtypeStruct((_NC * _NPAD,), jnp.float32),
        ),
        scratch_types=[
            pltpu.VMEM((_CPT, _CH), jnp.int32),
            pltpu.VMEM((_CPT, _CH), jnp.int32),
            pltpu.VMEM((_CPT, _CH), jnp.float32),
            pltpu.VMEM((_CH, _D), jnp.float32),
            pltpu.VMEM((_DWPT,), jnp.float32),
            pltpu.VMEM_SHARED((_NROW, _D), jnp.float32),
            pltpu.VMEM_SHARED((_NPAD,), jnp.float32),
            pltpu.SemaphoreType.DMA,
            pltpu.SemaphoreType.DMA,
        ],
    )
    def k(x_hbm, src_hbm, dst_hbm, w_hbm, out_hbm, dw_hbm,
          src_v, dst_v, w_v, rows_v, zdw_v, acc_s, dw_s, sem, dwsem):
        cid = lax.axis_index("c")
        sid = lax.axis_index("s")
        wid = cid * _NS + sid

        zero16 = jnp.zeros((16,), jnp.float32)

        def zrow(r, c):
            for j in range(_D // 16):
                rows_v[r, pl.ds(j * 16, 16)] = zero16
            return c
        lax.fori_loop(0, _ZR, zrow, 0)

        def zdw(i, c):
            zdw_v[pl.ds(i * 16, 16)] = zero16
            return c
        lax.fori_loop(0, _DWPT // 16, zdw, 0)

        for t in range(_RPT // _ZR):
            pltpu.sync_copy(rows_v, acc_s.at[pl.ds(sid * _RPT + t * _ZR, _ZR)])
        pltpu.sync_copy(zdw_v, dw_s.at[pl.ds(sid * _DWPT, _DWPT)])
        plsc.subcore_barrier()

        pltpu.sync_copy(src_hbm.at[wid], src_v)
        pltpu.sync_copy(dst_hbm.at[wid], dst_v)
        pltpu.sync_copy(w_hbm.at[wid], w_v)

        def chunk(i, c):
            pltpu.async_copy(x_hbm.at[src_v.at[i]], rows_v, sem).wait()

            def grp(g, c2):
                wv = w_v[i, pl.ds(g * 16, 16)]
                for kk in range(16):
                    ws = wv[kk]
                    e = g * 16 + kk
                    for j in range(_D // 16):
                        sl = pl.ds(j * 16, 16)
                        rows_v[e, sl] = rows_v[e, sl] * ws
                return c2
            lax.fori_loop(0, _CH // 16, grp, 0)

            pltpu.async_copy(w_v.at[i], dw_s.at[dst_v.at[i]], dwsem, add=True)
            pltpu.sync_copy(rows_v, acc_s.at[dst_v.at[i]], add=True)
            return c
        lax.fori_loop(0, _CPT, chunk, 0)

        def dwdrain(i, c):
            pltpu.make_async_copy(
                w_v.at[i], dw_s.at[dst_v.at[i]], dwsem).wait()
            return c
        lax.fori_loop(0, _CPT, dwdrain, 0)

        plsc.subcore_barrier()
        pltpu.sync_copy(acc_s.at[pl.ds(sid * _RPT, _RPT)],
                        out_hbm.at[cid, pl.ds(sid * _RPT, _RPT)])
        pltpu.sync_copy(dw_s.at[pl.ds(sid * _DWPT, _DWPT)],
                        dw_hbm.at[pl.ds(cid * _NPAD + sid * _DWPT, _DWPT)])

    return k(x, src_t, dst_t, w_t)


def _combine(p0, p1, dw0, dw1, xin, lin_W, lin_b, ln_g, ln_bt,
             fc_W=None, fc_b=None):
    bN = 1000
    final = fc_W is not None

    def kern(*refs):
        if final:
            (p0_ref, p1_ref, dw0_ref, dw1_ref, x_ref, w_ref, b_ref,
             g_ref, bt_ref, fw_ref, fb_ref, out_ref) = refs
        else:
            (p0_ref, p1_ref, dw0_ref, dw1_ref, x_ref, w_ref, b_ref,
             g_ref, bt_ref, out_ref) = refs
        dw = dw0_ref[...] + dw1_ref[...]
        aggr = p0_ref[...] + p1_ref[...] - dw * x_ref[...]
        h = lax.dot_general(aggr, w_ref[...], (((1,), (1,)), ((), ())),
                            preferred_element_type=jnp.float32) + b_ref[...]
        h = jnp.maximum(h, 0.0)
        mu = jnp.mean(h, axis=1, keepdims=True)
        hc = h - mu
        var = jnp.mean(hc * hc, axis=1, keepdims=True)
        hn = hc * lax.rsqrt(var + _EPS) * g_ref[...] + bt_ref[...]
        if final:
            hn = lax.dot_general(hn, fw_ref[...], (((1,), (1,)), ((), ())),
                                 preferred_element_type=jnp.float32) + fb_ref[...]
        out_ref[...] = hn

    row = pl.BlockSpec((bN, _D), lambda i: (i, 0))
    col = pl.BlockSpec((bN, 1), lambda i: (i, 0))
    full = pl.BlockSpec((_D, _D), lambda i: (0, 0))
    vec = pl.BlockSpec((1, _D), lambda i: (0, 0))
    in_specs = [row, row, col, col, row, full, vec, vec, vec]
    args = [p0, p1, dw0, dw1, xin, lin_W, lin_b.reshape(1, _D),
            ln_g.reshape(1, _D), ln_bt.reshape(1, _D)]
    if final:
        in_specs += [full, vec]
        args += [fc_W, fc_b.reshape(1, _D)]

    return pl.pallas_call(
        kern,
        grid=(_N // bN,),
        in_specs=in_specs,
        out_specs=row,
        out_shape=jax.ShapeDtypeStruct((_N, _D), jnp.float32),
    )(*args)


def kernel(x, edge_index, edge_attr, lin0_W, lin0_b, emlp0_W, emlp0_b,
           ln0_g, ln0_bt, lin1_W, lin1_b, emlp1_W, emlp1_b, ln1_g, ln1_bt,
           fc_W, fc_b):
    src = edge_index[0]
    dst = edge_index[1]

    w01 = _edge_weights(edge_attr, emlp0_W, emlp0_b, emlp1_W, emlp1_b)

    pad = _EPAD - _E
    src_t = jnp.pad(src, (0, pad)).reshape(_NT, _CPT, _CH)
    dst_t = jnp.pad(dst, (0, pad)).reshape(_NT, _CPT, _CH)
    w0_t = jnp.pad(w01[:, 0], (0, pad)).reshape(_NT, _CPT, _CH)
    w1_t = jnp.pad(w01[:, 1], (0, pad)).reshape(_NT, _CPT, _CH)

    p, dwp = _sc_scatter(x, src_t, dst_t, w0_t)
    dwp = dwp.reshape(_NC, _NPAD)
    dw0 = dwp[0, :_N].reshape(_N, 1)
    dw1 = dwp[1, :_N].reshape(_N, 1)
    h = _combine(p[0, :_N], p[1, :_N], dw0, dw1, x,
                 lin0_W, lin0_b, ln0_g, ln0_bt)

    p, dwp = _sc_scatter(h, src_t, dst_t, w1_t)
    dwp = dwp.reshape(_NC, _NPAD)
    dw0 = dwp[0, :_N].reshape(_N, 1)
    dw1 = dwp[1, :_N].reshape(_N, 1)
    return _combine(p[0, :_N], p[1, :_N], dw0, dw1, h,
                    lin1_W, lin1_b, ln1_g, ln1_bt, fc_W, fc_b)


# speedup vs baseline: 1.4592x; 1.0089x over previous
import functools

import jax
import jax.numpy as jnp
from jax import lax
from jax.experimental import pallas as pl
from jax.experimental.pallas import tpu as pltpu
from jax.experimental.pallas import tpu_sc as plsc

_N = 10000
_D = 128
_E = 320000
_EPS = 1e-5

_NC = 2
_NS = 16
_NT = _NC * _NS
_CH = 128
_CPT = -(-_E // (_NT * _CH))
_EPT = _CPT * _CH
_EPAD = _NT * _EPT
_NROW = 10240
_RPT = _NROW // _NS
_ZR = 128
_NPAD = 10240
_DWPT = _NPAD // _NS


def _edge_weights(edge_attr, w0, b0, w1, b1):
    bE = 10000

    def kern(ea_ref, w0_ref, b0_ref, w1_ref, b1_ref, out_ref):
        ea = ea_ref[...]
        z0 = jnp.dot(ea, w0_ref[...], preferred_element_type=jnp.float32) + b0_ref[...]
        z1 = jnp.dot(ea, w1_ref[...], preferred_element_type=jnp.float32) + b1_ref[...]
        z = jnp.concatenate([z0, z1], axis=1)
        out_ref[...] = jnp.maximum(z, 0.0) + jnp.log1p(jnp.exp(-jnp.abs(z)))

    return pl.pallas_call(
        kern,
        grid=(_E // bE,),
        in_specs=[
            pl.BlockSpec((bE, 16), lambda i: (i, 0)),
            pl.BlockSpec((16, 1), lambda i: (0, 0)),
            pl.BlockSpec((1, 1), lambda i: (0, 0)),
            pl.BlockSpec((16, 1), lambda i: (0, 0)),
            pl.BlockSpec((1, 1), lambda i: (0, 0)),
        ],
        out_specs=pl.BlockSpec((bE, 2), lambda i: (i, 0)),
        out_shape=jax.ShapeDtypeStruct((_E, 2), jnp.float32),
    )(edge_attr, w0, b0.reshape(1, 1), w1, b1.reshape(1, 1))


def _sc_scatter(x, src_t, dst_t, w_t):
    mesh = plsc.VectorSubcoreMesh(core_axis_name="c", subcore_axis_name="s")

    @functools.partial(
        pl.kernel,
        mesh=mesh,
        out_type=(
            jax.ShapeDtypeStruct((_NC, _NROW, _D), jnp.float32),
            jax.ShapeDtypeStruct((_NC * _NPAD,), jnp.float32),
        ),
        scratch_types=[
            pltpu.VMEM((_CPT, _CH), jnp.int32),
            pltpu.VMEM((_CPT, _CH), jnp.int32),
            pltpu.VMEM((_CPT, _CH), jnp.float32),
            pltpu.VMEM((_CH, _D), jnp.float32),
            pltpu.VMEM((_DWPT,), jnp.float32),
            pltpu.VMEM_SHARED((_NROW, _D), jnp.float32),
            pltpu.VMEM_SHARED((_NPAD,), jnp.float32),
            pltpu.SemaphoreType.DMA,
            pltpu.SemaphoreType.DMA,
        ],
    )
    def k(x_hbm, src_hbm, dst_hbm, w_hbm, out_hbm, dw_hbm,
          src_v, dst_v, w_v, rows_v, zdw_v, acc_s, dw_s, sem, dwsem):
        cid = lax.axis_index("c")
        sid = lax.axis_index("s")
        wid = cid * _NS + sid

        zero16 = jnp.zeros((16,), jnp.float32)

        def zrow(r, c):
            for j in range(_D // 16):
                rows_v[r, pl.ds(j * 16, 16)] = zero16
            return c
        lax.fori_loop(0, _ZR, zrow, 0)

        def zdw(i, c):
            zdw_v[pl.ds(i * 16, 16)] = zero16
            return c
        lax.fori_loop(0, _DWPT // 16, zdw, 0)

        for t in range(_RPT // _ZR):
            pltpu.async_copy(rows_v,
                             acc_s.at[pl.ds(sid * _RPT + t * _ZR, _ZR)], sem)
        pltpu.async_copy(zdw_v, dw_s.at[pl.ds(sid * _DWPT, _DWPT)], sem)
        pltpu.async_copy(src_hbm.at[wid], src_v, dwsem)
        pltpu.async_copy(dst_hbm.at[wid], dst_v, dwsem)
        pltpu.async_copy(w_hbm.at[wid], w_v, dwsem)
        for t in range(_RPT // _ZR):
            pltpu.make_async_copy(
                rows_v, acc_s.at[pl.ds(sid * _RPT, _ZR)], sem).wait()
        pltpu.make_async_copy(
            zdw_v, dw_s.at[pl.ds(sid * _DWPT, _DWPT)], sem).wait()
        pltpu.make_async_copy(src_hbm.at[wid], src_v, dwsem).wait()
        pltpu.make_async_copy(dst_hbm.at[wid], dst_v, dwsem).wait()
        pltpu.make_async_copy(w_hbm.at[wid], w_v, dwsem).wait()
        plsc.subcore_barrier()

        def chunk(i, c):
            pltpu.async_copy(x_hbm.at[src_v.at[i]], rows_v, sem).wait()
            pltpu.async_copy(w_v.at[i], dw_s.at[dst_v.at[i]], dwsem, add=True)

            def grp(g, c2):
                wv = w_v[i, pl.ds(g * 16, 16)]
                for kk in range(16):
                    ws = wv[kk]
                    e = g * 16 + kk
                    for j in range(_D // 16):
                        sl = pl.ds(j * 16, 16)
                        rows_v[e, sl] = rows_v[e, sl] * ws
                return c2
            lax.fori_loop(0, _CH // 16, grp, 0)

            pltpu.sync_copy(rows_v, acc_s.at[dst_v.at[i]], add=True)
            return c
        lax.fori_loop(0, _CPT, chunk, 0)

        def dwdrain(i, c):
            pltpu.make_async_copy(
                w_v.at[i], dw_s.at[dst_v.at[i]], dwsem).wait()
            return c
        lax.fori_loop(0, _CPT, dwdrain, 0)

        plsc.subcore_barrier()
        pltpu.sync_copy(acc_s.at[pl.ds(sid * _RPT, _RPT)],
                        out_hbm.at[cid, pl.ds(sid * _RPT, _RPT)])
        pltpu.sync_copy(dw_s.at[pl.ds(sid * _DWPT, _DWPT)],
                        dw_hbm.at[pl.ds(cid * _NPAD + sid * _DWPT, _DWPT)])

    return k(x, src_t, dst_t, w_t)


def _combine(p0, p1, dw0, dw1, xin, lin_W, lin_b, ln_g, ln_bt,
             fc_W=None, fc_b=None):
    bN = 1000
    final = fc_W is not None

    def kern(*refs):
        if final:
            (p0_ref, p1_ref, dw0_ref, dw1_ref, x_ref, w_ref, b_ref,
             g_ref, bt_ref, fw_ref, fb_ref, out_ref) = refs
        else:
            (p0_ref, p1_ref, dw0_ref, dw1_ref, x_ref, w_ref, b_ref,
             g_ref, bt_ref, out_ref) = refs
        dw = dw0_ref[...] + dw1_ref[...]
        aggr = p0_ref[...] + p1_ref[...] - dw * x_ref[...]
        h = lax.dot_general(aggr, w_ref[...], (((1,), (1,)), ((), ())),
                            preferred_element_type=jnp.float32) + b_ref[...]
        h = jnp.maximum(h, 0.0)
        mu = jnp.mean(h, axis=1, keepdims=True)
        hc = h - mu
        var = jnp.mean(hc * hc, axis=1, keepdims=True)
        hn = hc * lax.rsqrt(var + _EPS) * g_ref[...] + bt_ref[...]
        if final:
            hn = lax.dot_general(hn, fw_ref[...], (((1,), (1,)), ((), ())),
                                 preferred_element_type=jnp.float32) + fb_ref[...]
        out_ref[...] = hn

    row = pl.BlockSpec((bN, _D), lambda i: (i, 0))
    col = pl.BlockSpec((bN, 1), lambda i: (i, 0))
    full = pl.BlockSpec((_D, _D), lambda i: (0, 0))
    vec = pl.BlockSpec((1, _D), lambda i: (0, 0))
    in_specs = [row, row, col, col, row, full, vec, vec, vec]
    args = [p0, p1, dw0, dw1, xin, lin_W, lin_b.reshape(1, _D),
            ln_g.reshape(1, _D), ln_bt.reshape(1, _D)]
    if final:
        in_specs += [full, vec]
        args += [fc_W, fc_b.reshape(1, _D)]

    return pl.pallas_call(
        kern,
        grid=(_N // bN,),
        in_specs=in_specs,
        out_specs=row,
        out_shape=jax.ShapeDtypeStruct((_N, _D), jnp.float32),
    )(*args)


def kernel(x, edge_index, edge_attr, lin0_W, lin0_b, emlp0_W, emlp0_b,
           ln0_g, ln0_bt, lin1_W, lin1_b, emlp1_W, emlp1_b, ln1_g, ln1_bt,
           fc_W, fc_b):
    src = edge_index[0]
    dst = edge_index[1]

    w01 = _edge_weights(edge_attr, emlp0_W, emlp0_b, emlp1_W, emlp1_b)

    pad = _EPAD - _E
    src_t = jnp.pad(src, (0, pad)).reshape(_NT, _CPT, _CH)
    dst_t = jnp.pad(dst, (0, pad)).reshape(_NT, _CPT, _CH)
    w0_t = jnp.pad(w01[:, 0], (0, pad)).reshape(_NT, _CPT, _CH)
    w1_t = jnp.pad(w01[:, 1], (0, pad)).reshape(_NT, _CPT, _CH)

    p, dwp = _sc_scatter(x, src_t, dst_t, w0_t)
    dwp = dwp.reshape(_NC, _NPAD)
    dw0 = dwp[0, :_N].reshape(_N, 1)
    dw1 = dwp[1, :_N].reshape(_N, 1)
    h = _combine(p[0, :_N], p[1, :_N], dw0, dw1, x,
                 lin0_W, lin0_b, ln0_g, ln0_bt)

    p, dwp = _sc_scatter(h, src_t, dst_t, w1_t)
    dwp = dwp.reshape(_NC, _NPAD)
    dw0 = dwp[0, :_N].reshape(_N, 1)
    dw1 = dwp[1, :_N].reshape(_N, 1)
    return _combine(p[0, :_N], p[1, :_N], dw0, dw1, h,
                    lin1_W, lin1_b, ln1_g, ln1_bt, fc_W, fc_b)
